# Initial kernel scaffold; baseline (speedup 1.0000x reference)
import functools

import jax
import jax.numpy as jnp
from jax import lax
from jax.experimental import pallas as pl
from jax.experimental.pallas import tpu as pltpu
from jax.experimental.pallas import tpu_sc as plsc

_EPS = 1e-5
_NC = 2
_NS = 16
_NW = _NC * _NS
_CH = 128


def _sc_scatter_fn(npad, epad, d):
    ept = epad // _NW
    nch = ept // _CH
    rows_pt = npad // _NS
    ncopy = rows_pt // _CH

    mesh = plsc.VectorSubcoreMesh(core_axis_name="c", subcore_axis_name="s")

    def body(t_hbm, src_hbm, dst_hbm, zeros_hbm, out_hbm,
             idx_s, idx_d, rows, acc, sem):
        c = lax.axis_index("c")
        s = lax.axis_index("s")
        wid = s * _NC + c
        pltpu.sync_copy(zeros_hbm, acc.at[pl.ds(s * rows_pt, rows_pt)])
        pltpu.sync_copy(src_hbm.at[wid], idx_s)
        pltpu.sync_copy(dst_hbm.at[wid], idx_d)
        plsc.subcore_barrier()

        def step(j, carry):
            pltpu.async_copy(t_hbm.at[idx_s.at[j]], rows, sem).wait()
            pltpu.sync_copy(rows, acc.at[idx_d.at[j]], add=True)
            return carry

        lax.fori_loop(0, nch, step, 0)
        plsc.subcore_barrier()
        base = s * rows_pt
        for r in range(ncopy):
            pltpu.sync_copy(acc.at[pl.ds(base + r * _CH, _CH)], rows)
            pltpu.sync_copy(rows, out_hbm.at[c, pl.ds(base + r * _CH, _CH)])

    return pl.kernel(
        body,
        out_type=jax.ShapeDtypeStruct((_NC, npad, d), jnp.float32),
        mesh=mesh,
        scratch_types=[
            pltpu.VMEM((nch, _CH), jnp.int32),
            pltpu.VMEM((nch, _CH), jnp.int32),
            pltpu.VMEM((_CH, d), jnp.float32),
            pltpu.VMEM_SHARED((npad, d), jnp.float32),
            pltpu.SemaphoreType.DMA,
        ],
    )


def _tc_matmul(x, w, r=1024):
    npad, k = x.shape
    cout = w.shape[1]

    def body(x_ref, w_ref, o_ref):
        o_ref[...] = jnp.dot(x_ref[...], w_ref[...],
                             preferred_element_type=jnp.float32)

    return pl.pallas_call(
        body,
        grid=(npad // r,),
        in_specs=[pl.BlockSpec((r, k), lambda i: (i, 0)),
                  pl.BlockSpec((k, cout), lambda i: (0, 0))],
        out_specs=pl.BlockSpec((r, cout), lambda i: (i, 0)),
        out_shape=jax.ShapeDtypeStruct((npad, cout), jnp.float32),
    )(x, w)


def _tc_dinv_t0(degp, z0, n_valid, r=1024):
    npad, d0 = z0.shape

    def body(degp_ref, z_ref, dinv_ref, t_ref):
        i = pl.program_id(0)
        deg = degp_ref[0, :, 0:1] + degp_ref[1, :, 0:1] + 2.0
        rowid = lax.broadcasted_iota(jnp.int32, (r, 1), 0)
        valid = (rowid < (n_valid - i * r)).astype(jnp.float32)
        dcol = lax.rsqrt(deg) * valid
        dinv_ref[...] = jnp.broadcast_to(dcol, (r, 128))
        t_ref[...] = dcol * z_ref[...]

    return pl.pallas_call(
        body,
        grid=(npad // r,),
        in_specs=[pl.BlockSpec((2, r, 8), lambda i: (0, i, 0)),
                  pl.BlockSpec((r, d0), lambda i: (i, 0))],
        out_specs=[pl.BlockSpec((r, 128), lambda i: (i, 0)),
                   pl.BlockSpec((r, d0), lambda i: (i, 0))],
        out_shape=[jax.ShapeDtypeStruct((npad, 128), jnp.float32),
                   jax.ShapeDtypeStruct((npad, d0), jnp.float32)],
    )(degp, z0)


def _tc_layer(accp, t, dinv, w, b, n_valid, r=512):
    npad, din = t.shape
    cout = din if w is None else w.shape[1]
    grid = npad // r

    def body(*refs):
        if w is None:
            acc_ref, t_ref, dinv_ref, b_ref, y_ref, st_ref = refs
            w_ref = None
        else:
            acc_ref, t_ref, dinv_ref, w_ref, b_ref, y_ref, st_ref = refs
        i = pl.program_id(0)
        dv = dinv_ref[:, :din]
        agg = dv * (acc_ref[0] + acc_ref[1] + 2.0 * t_ref[...])
        if w_ref is None:
            z = agg
        else:
            z = jnp.dot(agg, w_ref[...], preferred_element_type=jnp.float32)
        y = jnp.maximum(z + b_ref[...], 0.0)
        y_ref[...] = y
        rowid = lax.broadcasted_iota(jnp.int32, (r, cout), 0)
        mask = (rowid < (n_valid - i * r)).astype(jnp.float32)
        ym = y * mask
        s1 = jnp.sum(ym, axis=0, keepdims=True)
        s2 = jnp.sum(ym * ym, axis=0, keepdims=True)
        contrib = jnp.concatenate(
            [s1, s2, jnp.zeros((6, cout), jnp.float32)], axis=0)

        @pl.when(i == 0)
        def _():
            st_ref[...] = contrib

        @pl.when(i > 0)
        def _():
            st_ref[...] = st_ref[...] + contrib

    in_specs = [pl.BlockSpec((2, r, din), lambda i: (0, i, 0)),
                pl.BlockSpec((r, din), lambda i: (i, 0)),
                pl.BlockSpec((r, 128), lambda i: (i, 0))]
    args = [accp, t, dinv]
    if w is not None:
        in_specs.append(pl.BlockSpec((din, cout), lambda i: (0, 0)))
        args.append(w)
    in_specs.append(pl.BlockSpec((1, cout), lambda i: (0, 0)))
    args.append(b)
    return pl.pallas_call(
        body,
        grid=(grid,),
        in_specs=in_specs,
        out_specs=[pl.BlockSpec((r, cout), lambda i: (i, 0)),
                   pl.BlockSpec((8, cout), lambda i: (0, 0))],
        out_shape=[jax.ShapeDtypeStruct((npad, cout), jnp.float32),
                   jax.ShapeDtypeStruct((8, cout), jnp.float32)],
    )(*args)


def _tc_bn_scale(y, st, g, be, dinv, n_valid, r=1024):
    npad, cout = y.shape
    inv_n = 1.0 / n_valid

    def body(y_ref, st_ref, g_ref, be_ref, dinv_ref, t_ref):
        m = st_ref[0:1, :] * inv_n
        v = st_ref[1:2, :] * inv_n - m * m
        sc = g_ref[...] * lax.rsqrt(v + _EPS)
        tt = be_ref[...] - m * sc
        t_ref[...] = dinv_ref[:, :cout] * (y_ref[...] * sc + tt)

    return pl.pallas_call(
        body,
        grid=(npad // r,),
        in_specs=[pl.BlockSpec((r, cout), lambda i: (i, 0)),
                  pl.BlockSpec((8, cout), lambda i: (0, 0)),
                  pl.BlockSpec((1, cout), lambda i: (0, 0)),
                  pl.BlockSpec((1, cout), lambda i: (0, 0)),
                  pl.BlockSpec((r, 128), lambda i: (i, 0))],
        out_specs=pl.BlockSpec((r, cout), lambda i: (i, 0)),
        out_shape=jax.ShapeDtypeStruct((npad, cout), jnp.float32),
    )(y, st, g, be, dinv)


def _tc_final(y4, st4, g4, be4, batchp, mlp, n_valid, n_graphs=8, r=512):
    npad, c = y4.shape
    grid = npad // r
    inv_n = 1.0 / n_valid
    (mw0, mb0, mg0, mbe0, mw1, mb1, mg1, mbe1,
     mw2, mb2, mg2, mbe2, fw, fb) = mlp
    n_classes = fw.shape[1]

    def _bn8(x, g_ref, be_ref):
        m = jnp.mean(x, axis=0, keepdims=True)
        v = jnp.mean((x - m) * (x - m), axis=0, keepdims=True)
        return (x - m) * lax.rsqrt(v + _EPS) * g_ref[...] + be_ref[...]

    def body(y_ref, st_ref, g_ref, be_ref, b_ref,
             mw0_r, mb0_r, mg0_r, mbe0_r, mw1_r, mb1_r, mg1_r, mbe1_r,
             mw2_r, mb2_r, mg2_r, mbe2_r, fw_r, fb_r, o_ref, pooled):
        i = pl.program_id(0)
        m = st_ref[0:1, :] * inv_n
        v = st_ref[1:2, :] * inv_n - m * m
        sc = g_ref[...] * lax.rsqrt(v + _EPS)
        tt = be_ref[...] - m * sc
        h = y_ref[...] * sc + tt
        bvec = b_ref[...]
        rowid = lax.broadcasted_iota(jnp.int32, (r, 1), 0)
        validrow = rowid < (n_valid - i * r)

        @pl.when(i == 0)
        def _():
            pooled[...] = jnp.full((n_graphs, c), -jnp.inf, jnp.float32)

        for gph in range(n_graphs):
            mk = jnp.logical_and(bvec == gph, validrow)
            mg_ = jnp.max(jnp.where(mk, h, -jnp.inf), axis=0, keepdims=True)
            pooled[gph:gph + 1, :] = jnp.maximum(pooled[gph:gph + 1, :], mg_)

        @pl.when(i == grid - 1)
        def _():
            p = pooled[...]
            for (w_r, bb_r, gg_r, bbe_r) in (
                    (mw0_r, mb0_r, mg0_r, mbe0_r),
                    (mw1_r, mb1_r, mg1_r, mbe1_r),
                    (mw2_r, mb2_r, mg2_r, mbe2_r)):
                z = jnp.dot(p, w_r[...],
                            preferred_element_type=jnp.float32) + bb_r[...]
                p = _bn8(jnp.maximum(z, 0.0), gg_r, bbe_r)
            o_ref[...] = jnp.dot(p, fw_r[...],
                                 preferred_element_type=jnp.float32) + fb_r[...]

    def _full(a):
        nd = len(a.shape)
        return pl.BlockSpec(a.shape, lambda i, _nd=nd: (0,) * _nd)

    in_specs = [pl.BlockSpec((r, c), lambda i: (i, 0)),
                pl.BlockSpec((8, c), lambda i: (0, 0)),
                pl.BlockSpec((1, c), lambda i: (0, 0)),
                pl.BlockSpec((1, c), lambda i: (0, 0)),
                pl.BlockSpec((r, 1), lambda i: (i, 0))]
    mlp_args = [mw0, mb0, mg0, mbe0, mw1, mb1, mg1, mbe1,
                mw2, mb2, mg2, mbe2, fw, fb]
    in_specs += [_full(a) for a in mlp_args]
    return pl.pallas_call(
        body,
        grid=(grid,),
        in_specs=in_specs,
        out_specs=pl.BlockSpec((n_graphs, n_classes), lambda i: (0, 0)),
        out_shape=jax.ShapeDtypeStruct((n_graphs, n_classes), jnp.float32),
        scratch_shapes=[pltpu.VMEM((n_graphs, c), jnp.float32)],
    )(y4, st4, g4, be4, batchp, *mlp_args)


def _ceil_to(a, m):
    return -(-a // m) * m


def kernel(x, edge_index, batch, W0, b0, g0, be0, W1, b1, g1, be1,
           W2, b2, g2, be2, W3, b3, g3, be3, W4, b4, g4, be4,
           mW0, mb0, mg0, mbe0, mW1, mb1, mg1, mbe1, mW2, mb2, mg2, mbe2,
           fW, fb):
    n, din = x.shape
    e = edge_index.shape[1]
    npad = _ceil_to(n + 1, _NS * _CH)
    epad = _ceil_to(e, _NW * _CH)

    xp = jnp.pad(x, ((0, npad - n), (0, 0)))
    srcp = jnp.concatenate(
        [edge_index[0], jnp.full((epad - e,), n, jnp.int32)]
    ).reshape(_NW, -1, _CH)
    dstp = jnp.concatenate(
        [edge_index[1], jnp.full((epad - e,), n, jnp.int32)]
    ).reshape(_NW, -1, _CH)
    batchp = jnp.pad(batch, (0, npad - n))[:, None]
    ones8 = jnp.ones((npad, 8), jnp.float32)
    zeros = {d: jnp.zeros((npad // _NS, d), jnp.float32) for d in (8, 64, 128)}

    degp = _sc_scatter_fn(npad, epad, 8)(ones8, dstp, dstp, zeros[8])
    z0 = _tc_matmul(xp, W0)
    dinv, t = _tc_dinv_t0(degp, z0, n)

    layers = [(None, b0, g0, be0), (W1, b1, g1, be1), (W2, b2, g2, be2),
              (W3, b3, g3, be3), (W4, b4, g4, be4)]
    y = st = None
    for i, (w, b, g, be) in enumerate(layers):
        d = t.shape[1]
        accp = _sc_scatter_fn(npad, epad, d)(t, srcp, dstp, zeros[d])
        y, st = _tc_layer(accp, t, dinv, w, b[None], n)
        if i < 4:
            t = _tc_bn_scale(y, st, g[None], be[None], dinv, n)

    mlp = (mW0, mb0[None], mg0[None], mbe0[None],
           mW1, mb1[None], mg1[None], mbe1[None],
           mW2, mb2[None], mg2[None], mbe2[None], fW, fb[None])
    return _tc_final(y, st, g4[None], be4[None], batchp, mlp, n)


# speedup vs baseline: 2.2187x; 2.2187x over previous
import jax
import jax.numpy as jnp
from jax import lax
from jax.experimental import pallas as pl
from jax.experimental.pallas import tpu as pltpu
from jax.experimental.pallas import tpu_sc as plsc

_EPS = 1e-5
_NC = 2
_NS = 16
_NW = _NC * _NS
_CH = 128


def _sc_scatter_fn(npad, epad, d, npanels=1, scaled=False):
    ept = epad // _NW
    nch = ept // _CH
    rows_pt = npad // _NS
    ncopy = rows_pt // _CH

    mesh = plsc.VectorSubcoreMesh(core_axis_name="c", subcore_axis_name="s")

    def body(*refs):
        if scaled:
            (t_hbm, src_hbm, dst_hbm, norm_hbm, zeros_hbm, out_hbm,
             idx_s, idx_d, norm_v, rows, acc, sem) = refs
        else:
            (t_hbm, src_hbm, dst_hbm, zeros_hbm, out_hbm,
             idx_s, idx_d, rows, acc, sem) = refs
        c = lax.axis_index("c")
        s = lax.axis_index("s")
        wid = s * _NC + c
        pltpu.sync_copy(src_hbm.at[wid], idx_s)
        pltpu.sync_copy(dst_hbm.at[wid], idx_d)
        if scaled:
            pltpu.sync_copy(norm_hbm.at[wid], norm_v)
        for p in range(npanels):
            pltpu.sync_copy(zeros_hbm, acc.at[pl.ds(s * rows_pt, rows_pt)])
            plsc.subcore_barrier()

            tp = t_hbm.at[p]

            def step(j, carry):
                pltpu.async_copy(tp.at[idx_s.at[j]], rows, sem).wait()
                if scaled:
                    def scale_kb(kb, cc):
                        nv = norm_v[j, pl.ds(16 * kb, 16)]
                        for k16 in range(16):
                            k = 16 * kb + k16
                            sk = nv[k16]
                            for q in range(d // 16):
                                rows[k, pl.ds(16 * q, 16)] = (
                                    rows[k, pl.ds(16 * q, 16)] * sk)
                        return cc

                    lax.fori_loop(0, _CH // 16, scale_kb, 0)
                pltpu.sync_copy(rows, acc.at[idx_d.at[j]], add=True)
                return carry

            lax.fori_loop(0, nch, step, 0)
            plsc.subcore_barrier()
            base = s * rows_pt
            for r in range(ncopy):
                pltpu.sync_copy(acc.at[pl.ds(base + r * _CH, _CH)], rows)
                pltpu.sync_copy(
                    rows, out_hbm.at[c, p, pl.ds(base + r * _CH, _CH)])

    scratch = [
        pltpu.VMEM((nch, _CH), jnp.int32),
        pltpu.VMEM((nch, _CH), jnp.int32),
    ]
    if scaled:
        scratch.append(pltpu.VMEM((nch, _CH), jnp.float32))
    scratch += [
        pltpu.VMEM((_CH, d), jnp.float32),
        pltpu.VMEM_SHARED((npad, d), jnp.float32),
        pltpu.SemaphoreType.DMA,
    ]
    return pl.kernel(
        body,
        out_type=jax.ShapeDtypeStruct((_NC, npanels, npad, d), jnp.float32),
        mesh=mesh,
        scratch_types=scratch,
        compiler_params=pltpu.CompilerParams(use_tc_tiling_on_sc=False,
                                             needs_layout_passes=False),
    )


def _sc_norm_fn(npad, epad):
    ept = epad // _NW
    nch = ept // _CH

    mesh = plsc.VectorSubcoreMesh(core_axis_name="c", subcore_axis_name="s")

    def body(dinv_hbm, src_hbm, dst_hbm, out_hbm,
             idx_s, idx_d, dinv_v, norm_v, sem):
        c = lax.axis_index("c")
        s = lax.axis_index("s")
        wid = s * _NC + c
        pltpu.sync_copy(src_hbm.at[wid], idx_s)
        pltpu.sync_copy(dst_hbm.at[wid], idx_d)
        pltpu.sync_copy(dinv_hbm, dinv_v)

        def step(j, carry):
            for kb in range(_CH // 16):
                s16 = idx_s[j, pl.ds(16 * kb, 16)]
                d16 = idx_d[j, pl.ds(16 * kb, 16)]
                dv_s = plsc.load_gather(dinv_v, [s16])
                dv_d = plsc.load_gather(dinv_v, [d16])
                norm_v[j, pl.ds(16 * kb, 16)] = dv_s * dv_d
            return carry

        lax.fori_loop(0, nch, step, 0)
        pltpu.sync_copy(norm_v, out_hbm.at[wid])

    return pl.kernel(
        body,
        out_type=jax.ShapeDtypeStruct((_NW, nch, _CH), jnp.float32),
        mesh=mesh,
        scratch_types=[
            pltpu.VMEM((nch, _CH), jnp.int32),
            pltpu.VMEM((nch, _CH), jnp.int32),
            pltpu.VMEM((npad,), jnp.float32),
            pltpu.VMEM((nch, _CH), jnp.float32),
            pltpu.SemaphoreType.DMA,
        ],
        compiler_params=pltpu.CompilerParams(use_tc_tiling_on_sc=False,
                                             needs_layout_passes=False),
    )


def _tc_dinv(degp, n_valid, r=1024):
    npad = degp.shape[1]

    def body(degp_ref, dinv_ref, dinv8_ref):
        i = pl.program_id(0)
        deg = degp_ref[0, :, 0:1] + degp_ref[1, :, 0:1] + 2.0
        rowid = lax.broadcasted_iota(jnp.int32, (r, 1), 0)
        valid = (rowid < (n_valid - i * r)).astype(jnp.float32)
        dcol = (1.0 / jnp.sqrt(deg)) * valid
        dinv_ref[...] = jnp.broadcast_to(dcol, (r, 128))
        dinv8_ref[...] = jnp.broadcast_to(dcol, (r, 8))

    return pl.pallas_call(
        body,
        grid=(npad // r,),
        in_specs=[pl.BlockSpec((2, r, 8), lambda i: (0, i, 0))],
        out_specs=[pl.BlockSpec((r, 128), lambda i: (i, 0)),
                   pl.BlockSpec((r, 8), lambda i: (i, 0))],
        out_shape=[jax.ShapeDtypeStruct((npad, 128), jnp.float32),
                   jax.ShapeDtypeStruct((npad, 8), jnp.float32)],
    )(degp)


def _tc_zt(y, st, stv, g, be, w, n_valid, prec=None, r=512):
    npad, cin = y.shape
    cout = w.shape[1]
    pd = min(cout, 128)
    npan = cout // pd
    with_bn = st is not None

    def body(*refs):
        if with_bn:
            y_ref, st_ref, stv_ref, g_ref, be_ref, w_ref, t_ref = refs
            m = st_ref[0:1, :] / n_valid
            v = stv_ref[0:1, :] / n_valid
            h = ((y_ref[...] - m) / jnp.sqrt(v + _EPS) * g_ref[...]
                 + be_ref[...])
        else:
            y_ref, w_ref, t_ref = refs
            h = y_ref[...]
        z = jnp.dot(h, w_ref[...], preferred_element_type=jnp.float32,
                    precision=prec)
        t_ref[...] = z[None]

    in_specs = [pl.BlockSpec((r, cin), lambda p, i: (i, 0))]
    args = [y]
    if with_bn:
        in_specs += [pl.BlockSpec((8, cin), lambda p, i: (0, 0)),
                     pl.BlockSpec((8, cin), lambda p, i: (0, 0)),
                     pl.BlockSpec((1, cin), lambda p, i: (0, 0)),
                     pl.BlockSpec((1, cin), lambda p, i: (0, 0))]
        args += [st, stv, g, be]
    in_specs += [pl.BlockSpec((cin, pd), lambda p, i: (0, p))]
    args += [w]
    return pl.pallas_call(
        body,
        grid=(npan, npad // r),
        in_specs=in_specs,
        out_specs=pl.BlockSpec((1, r, pd), lambda p, i: (p, i, 0)),
        out_shape=jax.ShapeDtypeStruct((npan, npad, pd), jnp.float32),
    )(*args)


def _tc_agg(accp, t, dinv, b, n_valid, r=512):
    npan, npad, pd = t.shape
    cout = npan * pd

    def body(acc_ref, t_ref, dinv_ref, b_ref, y_ref, st_ref):
        i = pl.program_id(1)
        dv = dinv_ref[:, 0:1]
        nl = dv * dv
        lt = t_ref[0] * nl
        y = jnp.maximum(
            ((acc_ref[0, 0] + acc_ref[1, 0]) + lt + lt) + b_ref[...], 0.0)
        y_ref[...] = y
        rowid = lax.broadcasted_iota(jnp.int32, (r, pd), 0)
        mask = (rowid < (n_valid - i * r)).astype(jnp.float32)
        ym = y * mask
        s1 = jnp.sum(ym, axis=0, keepdims=True)
        s2 = jnp.sum(ym * ym, axis=0, keepdims=True)
        contrib = jnp.concatenate(
            [s1, s2, jnp.zeros((6, pd), jnp.float32)], axis=0)

        @pl.when(i == 0)
        def _():
            st_ref[...] = contrib

        @pl.when(i > 0)
        def _():
            st_ref[...] = st_ref[...] + contrib

    return pl.pallas_call(
        body,
        grid=(npan, npad // r),
        in_specs=[pl.BlockSpec((2, 1, r, pd), lambda p, i: (0, p, i, 0)),
                  pl.BlockSpec((1, r, pd), lambda p, i: (p, i, 0)),
                  pl.BlockSpec((r, 128), lambda p, i: (i, 0)),
                  pl.BlockSpec((1, pd), lambda p, i: (0, p))],
        out_specs=[pl.BlockSpec((r, pd), lambda p, i: (i, p)),
                   pl.BlockSpec((8, pd), lambda p, i: (0, p))],
        out_shape=[jax.ShapeDtypeStruct((npad, cout), jnp.float32),
                   jax.ShapeDtypeStruct((8, cout), jnp.float32)],
    )(accp, t, dinv, b)


def _tc_var(y, st, n_valid, r=512):
    npad, cout = y.shape
    grid = npad // r

    def body(y_ref, st_ref, o_ref):
        i = pl.program_id(0)
        m = st_ref[0:1, :] / n_valid
        rowid = lax.broadcasted_iota(jnp.int32, (r, cout), 0)
        mask = rowid < (n_valid - i * r)
        d = jnp.where(mask, y_ref[...] - m, 0.0)
        contrib = jnp.concatenate(
            [jnp.sum(d * d, axis=0, keepdims=True),
             jnp.zeros((7, cout), jnp.float32)], axis=0)

        @pl.when(i == 0)
        def _():
            o_ref[...] = contrib

        @pl.when(i > 0)
        def _():
            o_ref[...] = o_ref[...] + contrib

    return pl.pallas_call(
        body,
        grid=(grid,),
        in_specs=[pl.BlockSpec((r, cout), lambda i: (i, 0)),
                  pl.BlockSpec((8, cout), lambda i: (0, 0))],
        out_specs=pl.BlockSpec((8, cout), lambda i: (0, 0)),
        out_shape=jax.ShapeDtypeStruct((8, cout), jnp.float32),
    )(y, st)


def _tc_final(y4, st4, stv4, g4, be4, batchp, mlp, n_valid, n_graphs=8,
              r=512):
    npad, c = y4.shape
    grid = npad // r
    inv_n = 1.0 / n_valid
    (mw0, mb0, mg0, mbe0, mw1, mb1, mg1, mbe1,
     mw2, mb2, mg2, mbe2, fw, fb) = mlp
    n_classes = fw.shape[1]

    def _bn8(x, g_ref, be_ref):
        m = jnp.mean(x, axis=0, keepdims=True)
        v = jnp.mean((x - m) * (x - m), axis=0, keepdims=True)
        return (x - m) / jnp.sqrt(v + _EPS) * g_ref[...] + be_ref[...]

    def body(y_ref, st_ref, stv_ref, g_ref, be_ref, b_ref,
             mw0_r, mb0_r, mg0_r, mbe0_r, mw1_r, mb1_r, mg1_r, mbe1_r,
             mw2_r, mb2_r, mg2_r, mbe2_r, fw_r, fb_r, o_ref, pooled):
        i = pl.program_id(0)
        m = st_ref[0:1, :] / n_valid
        v = stv_ref[0:1, :] / n_valid
        h = ((y_ref[...] - m) / jnp.sqrt(v + _EPS) * g_ref[...]
             + be_ref[...])
        bvec = b_ref[...]
        rowid = lax.broadcasted_iota(jnp.int32, (r, 1), 0)
        validrow = rowid < (n_valid - i * r)

        @pl.when(i == 0)
        def _():
            pooled[...] = jnp.full((n_graphs, c), -jnp.inf, jnp.float32)

        for gph in range(n_graphs):
            mk = jnp.logical_and(bvec == gph, validrow)
            mg_ = jnp.max(jnp.where(mk, h, -jnp.inf), axis=0, keepdims=True)
            pooled[gph:gph + 1, :] = jnp.maximum(pooled[gph:gph + 1, :], mg_)

        @pl.when(i == grid - 1)
        def _():
            p = pooled[...]
            for (w_r, bb_r, gg_r, bbe_r) in (
                    (mw0_r, mb0_r, mg0_r, mbe0_r),
                    (mw1_r, mb1_r, mg1_r, mbe1_r),
                    (mw2_r, mb2_r, mg2_r, mbe2_r)):
                z = jnp.dot(p, w_r[...],
                            preferred_element_type=jnp.float32) + bb_r[...]
                p = _bn8(jnp.maximum(z, 0.0), gg_r, bbe_r)
            o_ref[...] = jnp.dot(p, fw_r[...],
                                 preferred_element_type=jnp.float32) + fb_r[...]

    def _full(a):
        nd = len(a.shape)
        return pl.BlockSpec(a.shape, lambda i, _nd=nd: (0,) * _nd)

    in_specs = [pl.BlockSpec((r, c), lambda i: (i, 0)),
                pl.BlockSpec((8, c), lambda i: (0, 0)),
                pl.BlockSpec((8, c), lambda i: (0, 0)),
                pl.BlockSpec((1, c), lambda i: (0, 0)),
                pl.BlockSpec((1, c), lambda i: (0, 0)),
                pl.BlockSpec((r, 1), lambda i: (i, 0))]
    mlp_args = [mw0, mb0, mg0, mbe0, mw1, mb1, mg1, mbe1,
                mw2, mb2, mg2, mbe2, fw, fb]
    in_specs += [_full(a) for a in mlp_args]
    return pl.pallas_call(
        body,
        grid=(grid,),
        in_specs=in_specs,
        out_specs=pl.BlockSpec((n_graphs, n_classes), lambda i: (0, 0)),
        out_shape=jax.ShapeDtypeStruct((n_graphs, n_classes), jnp.float32),
        scratch_shapes=[pltpu.VMEM((n_graphs, c), jnp.float32)],
    )(y4, st4, stv4, g4, be4, batchp, *mlp_args)


def _ceil_to(a, m):
    return -(-a // m) * m


def kernel(x, edge_index, batch, W0, b0, g0, be0, W1, b1, g1, be1,
           W2, b2, g2, be2, W3, b3, g3, be3, W4, b4, g4, be4,
           mW0, mb0, mg0, mbe0, mW1, mb1, mg1, mbe1, mW2, mb2, mg2, mbe2,
           fW, fb):
    n, din = x.shape
    e = edge_index.shape[1]
    npad = _ceil_to(n + 1, _NS * _CH)
    epad = _ceil_to(e, _NW * _CH)

    xp = jnp.pad(x, ((0, npad - n), (0, 0)))
    perm = jnp.argsort(edge_index[1], stable=True)
    src_s = edge_index[0][perm]
    dst_s = edge_index[1][perm]
    srcp = jnp.concatenate(
        [src_s, jnp.full((epad - e,), n, jnp.int32)]).reshape(_NW, -1, _CH)
    dstp = jnp.concatenate(
        [dst_s, jnp.full((epad - e,), n, jnp.int32)]).reshape(_NW, -1, _CH)
    batchp = jnp.pad(batch, (0, npad - n))[:, None]
    ones8 = jnp.ones((npad, 8), jnp.float32)
    zeros = {d: jnp.zeros((npad // _NS, d), jnp.float32) for d in (8, 64, 128)}

    degp = _sc_scatter_fn(npad, epad, 8)(
        ones8.reshape(1, npad, 8), dstp, dstp, zeros[8])
    dinv, dinv8 = _tc_dinv(degp.reshape(_NC, npad, 8), n)
    normp = _sc_norm_fn(npad, epad)(dinv8[:, 0], srcp, dstp)

    layers = [(W0, b0, g0, be0), (W1, b1, g1, be1), (W2, b2, g2, be2),
              (W3, b3, g3, be3), (W4, b4, g4, be4)]
    y = st = stv = None
    g_prev = be_prev = None
    for i, (w, b, g, be) in enumerate(layers):
        if i == 0:
            t = _tc_zt(xp, None, None, None, None, w, n)
        else:
            t = _tc_zt(y, st, stv, g_prev[None], be_prev[None], w, n)
        npan, _, pd = t.shape
        accp = _sc_scatter_fn(npad, epad, pd, npan, scaled=True)(
            t, srcp, dstp, normp, zeros[pd])
        y, st = _tc_agg(accp, t, dinv, b[None], n)
        stv = _tc_var(y, st, n)
        g_prev, be_prev = g, be

    mlp = (mW0, mb0[None], mg0[None], mbe0[None],
           mW1, mb1[None], mg1[None], mbe1[None],
           mW2, mb2[None], mg2[None], mbe2[None], fW, fb[None])
    return _tc_final(y, st, stv, g4[None], be4[None], batchp, mlp, n)
